# baseline (speedup 1.0000x reference)
import functools

import jax
import jax.numpy as jnp
from jax.experimental import pallas as pl
from jax.experimental.pallas import tpu as pltpu

E = 8
D = 2048
H = 4096
T = 8192

TM = 1024
TH = 1024
CH = 512


def _mlp_body(x_ref, w1_ref, b1_ref, w2_ref, b2_ref, o_ref):
    h_id = pl.program_id(2)
    x = x_ref[...]
    acc = None
    for k in range(TH // CH):
        sl = slice(k * CH, (k + 1) * CH)
        hk = jnp.dot(x, w1_ref[0, :, sl], preferred_element_type=jnp.float32)
        hk = jnp.maximum(hk + b1_ref[0, :, sl], 0.0).astype(jnp.bfloat16)
        pk = jnp.dot(hk, w2_ref[0, sl, :], preferred_element_type=jnp.float32)
        acc = pk if acc is None else acc + pk

    @pl.when(h_id == 0)
    def _init():
        o_ref[0] = acc + b2_ref[0]

    @pl.when(h_id > 0)
    def _accum():
        o_ref[0] += acc


@functools.partial(jax.jit, static_argnames=("interpret",))
def kernel(x, W1, b1, W2, b2, interpret=False):
    e, d, h, t = W1.shape[0], x.shape[1], W1.shape[2], x.shape[0]
    xb = x.astype(jnp.bfloat16)
    W1b = W1.astype(jnp.bfloat16)
    W2b = W2.astype(jnp.bfloat16)
    b1r = b1.reshape(e, 1, h)
    b2r = b2.reshape(e, 1, d)
    grid = (t // TM, e, h // TH)
    return pl.pallas_call(
        _mlp_body,
        grid=grid,
        in_specs=[
            pl.BlockSpec((TM, d), lambda ti, ei, hi: (ti, 0)),
            pl.BlockSpec((1, d, TH), lambda ti, ei, hi: (ei, 0, hi)),
            pl.BlockSpec((1, 1, TH), lambda ti, ei, hi: (ei, 0, hi)),
            pl.BlockSpec((1, TH, d), lambda ti, ei, hi: (ei, hi, 0)),
            pl.BlockSpec((1, 1, d), lambda ti, ei, hi: (ei, 0, 0)),
        ],
        out_specs=pl.BlockSpec((1, TM, d), lambda ti, ei, hi: (ei, ti, 0)),
        out_shape=jax.ShapeDtypeStruct((e, t, d), jnp.float32),
        compiler_params=pltpu.CompilerParams(
            dimension_semantics=("parallel", "parallel", "arbitrary"),
        ),
        interpret=interpret,
    )(xb, W1b, b1r, W2b, b2r)


# speedup vs baseline: 1.0481x; 1.0481x over previous
import functools

import jax
import jax.numpy as jnp
from jax.experimental import pallas as pl
from jax.experimental.pallas import tpu as pltpu

E = 8
D = 2048
H = 4096
T = 8192

TM = 512
CH = 512


def _mlp_body(x_ref, w1_ref, b1_ref, w2_ref, b2_ref, o_ref):
    x = x_ref[...]
    acc = None
    for k in range(H // CH):
        sl = slice(k * CH, (k + 1) * CH)
        hk = jnp.dot(x, w1_ref[0, :, sl], preferred_element_type=jnp.float32)
        hk = jnp.maximum(hk + b1_ref[0, :, sl], 0.0).astype(jnp.bfloat16)
        pk = jnp.dot(hk, w2_ref[0, sl, :], preferred_element_type=jnp.float32)
        acc = pk if acc is None else acc + pk
    o_ref[0] = acc + b2_ref[0]


@functools.partial(jax.jit, static_argnames=("interpret",))
def kernel(x, W1, b1, W2, b2, interpret=False):
    e, d, h, t = W1.shape[0], x.shape[1], W1.shape[2], x.shape[0]
    xb = x.astype(jnp.bfloat16)
    W1b = W1.astype(jnp.bfloat16)
    W2b = W2.astype(jnp.bfloat16)
    b1r = b1.reshape(e, 1, h)
    b2r = b2.reshape(e, 1, d)
    grid = (e, t // TM)
    single = pl.Buffered(buffer_count=1)
    return pl.pallas_call(
        _mlp_body,
        grid=grid,
        in_specs=[
            pl.BlockSpec((TM, d), lambda ei, ti: (ti, 0)),
            pl.BlockSpec((1, d, h), lambda ei, ti: (ei, 0, 0), pipeline_mode=single),
            pl.BlockSpec((1, 1, h), lambda ei, ti: (ei, 0, 0)),
            pl.BlockSpec((1, h, d), lambda ei, ti: (ei, 0, 0), pipeline_mode=single),
            pl.BlockSpec((1, 1, d), lambda ei, ti: (ei, 0, 0)),
        ],
        out_specs=pl.BlockSpec((1, TM, d), lambda ei, ti: (ei, ti, 0)),
        out_shape=jax.ShapeDtypeStruct((e, t, d), jnp.float32),
        compiler_params=pltpu.CompilerParams(
            dimension_semantics=("arbitrary", "arbitrary"),
        ),
        interpret=interpret,
    )(xb, W1b, b1r, W2b, b2r)


# speedup vs baseline: 1.0505x; 1.0023x over previous
import functools

import jax
import jax.numpy as jnp
from jax.experimental import pallas as pl
from jax.experimental.pallas import tpu as pltpu

E = 8
D = 2048
H = 4096
T = 8192

TM = 512
CH = 1024


def _mlp_body(x_ref, w1_ref, b1_ref, w2_ref, b2_ref, o_ref):
    x = x_ref[...]
    acc = None
    for k in range(H // CH):
        sl = slice(k * CH, (k + 1) * CH)
        hk = jnp.dot(x, w1_ref[0, :, sl], preferred_element_type=jnp.float32)
        hk = jnp.maximum(hk + b1_ref[0, :, sl], 0.0).astype(jnp.bfloat16)
        pk = jnp.dot(hk, w2_ref[0, sl, :], preferred_element_type=jnp.float32)
        acc = pk if acc is None else acc + pk
    o_ref[0] = acc + b2_ref[0]


@functools.partial(jax.jit, static_argnames=("interpret",))
def kernel(x, W1, b1, W2, b2, interpret=False):
    e, d, h, t = W1.shape[0], x.shape[1], W1.shape[2], x.shape[0]
    xb = x.astype(jnp.bfloat16)
    W1b = W1.astype(jnp.bfloat16)
    W2b = W2.astype(jnp.bfloat16)
    b1r = b1.reshape(e, 1, h)
    b2r = b2.reshape(e, 1, d)
    grid = (e, t // TM)
    single = pl.Buffered(buffer_count=1)
    return pl.pallas_call(
        _mlp_body,
        grid=grid,
        in_specs=[
            pl.BlockSpec((TM, d), lambda ei, ti: (ti, 0)),
            pl.BlockSpec((1, d, h), lambda ei, ti: (ei, 0, 0), pipeline_mode=single),
            pl.BlockSpec((1, 1, h), lambda ei, ti: (ei, 0, 0)),
            pl.BlockSpec((1, h, d), lambda ei, ti: (ei, 0, 0), pipeline_mode=single),
            pl.BlockSpec((1, 1, d), lambda ei, ti: (ei, 0, 0)),
        ],
        out_specs=pl.BlockSpec((1, TM, d), lambda ei, ti: (ei, ti, 0)),
        out_shape=jax.ShapeDtypeStruct((e, t, d), jnp.float32),
        compiler_params=pltpu.CompilerParams(
            dimension_semantics=("arbitrary", "arbitrary"),
        ),
        interpret=interpret,
    )(xb, W1b, b1r, W2b, b2r)
